# Initial kernel scaffold; baseline (speedup 1.0000x reference)
import jax
import jax.numpy as jnp
from jax.experimental import pallas as pl


def kernel(x, edge_index, edge_type, weight1, root1, bias1, weight2, root2, bias2):
    raise NotImplementedError("write your pallas kernel here")


# speedup vs baseline: 401.9525x; 401.9525x over previous
import functools

import jax
import jax.numpy as jnp
from jax import lax
from jax.experimental import pallas as pl
from jax.experimental.pallas import tpu as pltpu
from jax.experimental.pallas import tpu_sc as plsc

N = 7
R = 16
E = 640000
HID = 16
OUT = 8
RS = R * N
BINS = N * RS
L = 16
NW = 32
EPW = E // NW
VECS = EPW // L


def _sc_hist_kernel(src_hbm, dst_hbm, typ_hbm, out_hbm,
                    src_v, dst_v, typ_v, hist_v, final_v,
                    sem0, sem1, sem2):
    wid = lax.axis_index("s") * 2 + lax.axis_index("c")
    base = wid * EPW
    cp0 = pltpu.async_copy(src_hbm.at[pl.ds(base, EPW)], src_v, sem0)
    cp1 = pltpu.async_copy(dst_hbm.at[pl.ds(base, EPW)], dst_v, sem1)
    cp2 = pltpu.async_copy(typ_hbm.at[pl.ds(base, EPW)], typ_v, sem2)

    zeros = jnp.zeros((L,), jnp.float32)

    def zbody(i, c):
        hist_v[pl.ds(i * L, L)] = zeros
        return c

    lax.fori_loop(0, BINS, zbody, 0)

    cp0.wait()
    cp1.wait()
    cp2.wait()

    lane_off = lax.broadcasted_iota(jnp.int32, (L,), 0) * BINS
    ones = jnp.ones((L,), jnp.float32)

    def body(i, c):
        o = i * L
        s = src_v[pl.ds(o, L)]
        d = dst_v[pl.ds(o, L)]
        t = typ_v[pl.ds(o, L)]
        key = d * RS + t * N + s
        plsc.addupdate_scatter(hist_v, [lane_off + key], ones)
        return c

    lax.fori_loop(0, VECS, body, 0)

    def rbody(i, c):
        o = i * L
        acc = hist_v[pl.ds(o, L)]
        for l in range(1, L):
            acc = acc + hist_v[pl.ds(l * BINS + o, L)]
        final_v[pl.ds(o, L)] = acc
        return c

    lax.fori_loop(0, BINS // L, rbody, 0)

    pltpu.sync_copy(final_v, out_hbm.at[wid])


def _make_sc_hist():
    return pl.kernel(
        _sc_hist_kernel,
        mesh=plsc.VectorSubcoreMesh(core_axis_name="c", subcore_axis_name="s"),
        out_type=jax.ShapeDtypeStruct((NW, BINS), jnp.float32),
        compiler_params=pltpu.CompilerParams(needs_layout_passes=False),
        scratch_types=[
            pltpu.VMEM((EPW,), jnp.int32),
            pltpu.VMEM((EPW,), jnp.int32),
            pltpu.VMEM((EPW,), jnp.int32),
            pltpu.VMEM((L * BINS,), jnp.float32),
            pltpu.VMEM((BINS,), jnp.float32),
            pltpu.SemaphoreType.DMA,
            pltpu.SemaphoreType.DMA,
            pltpu.SemaphoreType.DMA,
        ],
    )


def _tc_finish_body(parts_ref, w1_ref, r1_ref, b1_ref, w2_ref, rt2_ref,
                    b2_ref, out_ref):
    counts = jnp.sum(parts_ref[...], axis=0)
    g = (lax.broadcasted_iota(jnp.int32, (RS, R), 0) // N
         == lax.broadcasted_iota(jnp.int32, (RS, R), 1)).astype(jnp.float32)
    gt = (lax.broadcasted_iota(jnp.int32, (R, RS), 1) // N
          == lax.broadcasted_iota(jnp.int32, (R, RS), 0)).astype(jnp.float32)
    cnt = jnp.dot(counts, g, preferred_element_type=jnp.float32)
    denom = jnp.maximum(
        jnp.dot(cnt, gt, preferred_element_type=jnp.float32), 1.0)
    q = counts / denom
    agg1 = jnp.dot(q, w1_ref[...], preferred_element_type=jnp.float32)
    h = jnp.maximum(agg1 + r1_ref[...] + b1_ref[...], 0.0)
    w2h = jnp.concatenate(
        [jnp.dot(h, w2_ref[r], preferred_element_type=jnp.float32)
         for r in range(R)], axis=0)
    acc = (jnp.dot(q, w2h, preferred_element_type=jnp.float32)
           + jnp.dot(h, rt2_ref[...], preferred_element_type=jnp.float32)
           + b2_ref[...])
    m = jnp.max(acc, axis=1, keepdims=True)
    e = jnp.exp(acc - m)
    lse = jnp.log(jnp.sum(e, axis=1, keepdims=True))
    out_ref[...] = acc - m - lse


def kernel(x, edge_index, edge_type, weight1, root1, bias1, weight2, root2,
           bias2):
    del x
    src = edge_index[0]
    dst = edge_index[1]
    parts = _make_sc_hist()(src, dst, edge_type)
    parts = parts.reshape(NW, N, RS)
    return pl.pallas_call(
        _tc_finish_body,
        out_shape=jax.ShapeDtypeStruct((N, OUT), jnp.float32),
    )(parts, weight1.reshape(RS, HID), root1, bias1.reshape(1, HID),
      weight2, root2, bias2.reshape(1, OUT))


# speedup vs baseline: 465.6103x; 1.1584x over previous
import functools

import jax
import jax.numpy as jnp
from jax import lax
from jax.experimental import pallas as pl
from jax.experimental.pallas import tpu as pltpu
from jax.experimental.pallas import tpu_sc as plsc

N = 7
R = 16
E = 640000
HID = 16
OUT = 8
RS = R * N
BINS = N * RS
L = 16
NW = 32
EPW = E // NW
VECS = EPW // L


def _sc_hist_kernel(ei_hbm, typ_hbm, out_hbm,
                    src_v, dst_v, typ_v, hist_v, final_v,
                    sem0, sem1, sem2):
    wid = lax.axis_index("s") * 2 + lax.axis_index("c")
    base = wid * EPW
    cp0 = pltpu.async_copy(ei_hbm.at[pl.ds(base, EPW)], src_v, sem0)
    cp1 = pltpu.async_copy(ei_hbm.at[pl.ds(E + base, EPW)], dst_v, sem1)
    cp2 = pltpu.async_copy(typ_hbm.at[pl.ds(base, EPW)], typ_v, sem2)

    zeros = jnp.zeros((L,), jnp.float32)

    def zbody(i, c):
        hist_v[pl.ds(i * L, L)] = zeros
        return c

    lax.fori_loop(0, BINS, zbody, 0, unroll=8)

    cp0.wait()
    cp1.wait()
    cp2.wait()

    lane_off = lax.broadcasted_iota(jnp.int32, (L,), 0) * BINS
    ones = jnp.ones((L,), jnp.float32)

    def body(i, c):
        o = i * L
        s = src_v[pl.ds(o, L)]
        d = dst_v[pl.ds(o, L)]
        t = typ_v[pl.ds(o, L)]
        key = d * RS + t * N + s
        plsc.addupdate_scatter(hist_v, [lane_off + key], ones)
        return c

    lax.fori_loop(0, VECS, body, 0, unroll=8)

    def rbody(i, c):
        o = i * L
        acc = hist_v[pl.ds(o, L)]
        for l in range(1, L):
            acc = acc + hist_v[pl.ds(l * BINS + o, L)]
        final_v[pl.ds(o, L)] = acc
        return c

    lax.fori_loop(0, BINS // L, rbody, 0, unroll=2)

    pltpu.sync_copy(final_v, out_hbm.at[wid])


def _make_sc_hist():
    return pl.kernel(
        _sc_hist_kernel,
        mesh=plsc.VectorSubcoreMesh(core_axis_name="c", subcore_axis_name="s"),
        out_type=jax.ShapeDtypeStruct((NW, BINS), jnp.float32),
        compiler_params=pltpu.CompilerParams(needs_layout_passes=False),
        scratch_types=[
            pltpu.VMEM((EPW,), jnp.int32),
            pltpu.VMEM((EPW,), jnp.int32),
            pltpu.VMEM((EPW,), jnp.int32),
            pltpu.VMEM((L * BINS,), jnp.float32),
            pltpu.VMEM((BINS,), jnp.float32),
            pltpu.SemaphoreType.DMA,
            pltpu.SemaphoreType.DMA,
            pltpu.SemaphoreType.DMA,
        ],
    )


def _tc_finish_body(parts_ref, w1_ref, r1_ref, b1_ref, w2_ref, rt2_ref,
                    b2_ref, out_ref):
    counts = jnp.sum(parts_ref[...], axis=0)
    g = (lax.broadcasted_iota(jnp.int32, (RS, R), 0) // N
         == lax.broadcasted_iota(jnp.int32, (RS, R), 1)).astype(jnp.float32)
    gt = (lax.broadcasted_iota(jnp.int32, (R, RS), 1) // N
          == lax.broadcasted_iota(jnp.int32, (R, RS), 0)).astype(jnp.float32)
    cnt = jnp.dot(counts, g, preferred_element_type=jnp.float32)
    denom = jnp.maximum(
        jnp.dot(cnt, gt, preferred_element_type=jnp.float32), 1.0)
    q = counts / denom
    agg1 = jnp.dot(q, w1_ref[...], preferred_element_type=jnp.float32)
    h = jnp.maximum(agg1 + r1_ref[...] + b1_ref[...], 0.0)
    w2h = jnp.concatenate(
        [jnp.dot(h, w2_ref[r], preferred_element_type=jnp.float32)
         for r in range(R)], axis=0)
    acc = (jnp.dot(q, w2h, preferred_element_type=jnp.float32)
           + jnp.dot(h, rt2_ref[...], preferred_element_type=jnp.float32)
           + b2_ref[...])
    m = jnp.max(acc, axis=1, keepdims=True)
    e = jnp.exp(acc - m)
    lse = jnp.log(jnp.sum(e, axis=1, keepdims=True))
    out_ref[...] = acc - m - lse


def kernel(x, edge_index, edge_type, weight1, root1, bias1, weight2, root2,
           bias2):
    del x
    parts = _make_sc_hist()(edge_index.reshape(2 * E), edge_type)
    parts = parts.reshape(NW, N, RS)
    return pl.pallas_call(
        _tc_finish_body,
        out_shape=jax.ShapeDtypeStruct((N, OUT), jnp.float32),
    )(parts, weight1.reshape(RS, HID), root1, bias1.reshape(1, HID),
      weight2, root2, bias2.reshape(1, OUT))


# speedup vs baseline: 575.5337x; 1.2361x over previous
import functools

import jax
import jax.numpy as jnp
from jax import lax
from jax.experimental import pallas as pl
from jax.experimental.pallas import tpu as pltpu
from jax.experimental.pallas import tpu_sc as plsc

N = 7
R = 16
E = 640000
HID = 16
OUT = 8
RS = R * N
BINS = N * RS
L = 16
NW = 32
EPW = E // NW
VECS = EPW // L


def _sc_hist_kernel(ei_hbm, typ_hbm, out_hbm,
                    src_v, dst_v, typ_v, hist_v, final_v,
                    sem0, sem1, sem2):
    wid = lax.axis_index("s") * 2 + lax.axis_index("c")
    base = wid * EPW
    cp0 = pltpu.async_copy(ei_hbm.at[pl.ds(base, EPW)], src_v, sem0)
    cp1 = pltpu.async_copy(ei_hbm.at[pl.ds(E + base, EPW)], dst_v, sem1)
    cp2 = pltpu.async_copy(typ_hbm.at[pl.ds(base, EPW)], typ_v, sem2)

    zeros = jnp.zeros((L,), jnp.float32)

    def zbody(i, c):
        hist_v[pl.ds(i * L, L)] = zeros
        return c

    lax.fori_loop(0, BINS, zbody, 0, unroll=8)

    cp0.wait()
    cp1.wait()
    cp2.wait()

    lane_off = lax.broadcasted_iota(jnp.int32, (L,), 0) * BINS
    ones = jnp.ones((L,), jnp.float32)

    @plsc.parallel_loop(0, VECS, unroll=8)
    def _(i):
        o = i * L
        s = src_v[pl.ds(o, L)]
        d = dst_v[pl.ds(o, L)]
        t = typ_v[pl.ds(o, L)]
        plsc.addupdate_scatter(hist_v, [(d * RS + t * N) + (s + lane_off)],
                               ones)

    def rbody(i, c):
        o = i * L
        acc = hist_v[pl.ds(o, L)]
        for l in range(1, L):
            acc = acc + hist_v[pl.ds(l * BINS + o, L)]
        final_v[pl.ds(o, L)] = acc
        return c

    lax.fori_loop(0, BINS // L, rbody, 0, unroll=2)

    pltpu.sync_copy(final_v, out_hbm.at[wid])


def _make_sc_hist():
    return pl.kernel(
        _sc_hist_kernel,
        mesh=plsc.VectorSubcoreMesh(core_axis_name="c", subcore_axis_name="s"),
        out_type=jax.ShapeDtypeStruct((NW, BINS), jnp.float32),
        compiler_params=pltpu.CompilerParams(needs_layout_passes=False),
        scratch_types=[
            pltpu.VMEM((EPW,), jnp.int32),
            pltpu.VMEM((EPW,), jnp.int32),
            pltpu.VMEM((EPW,), jnp.int32),
            pltpu.VMEM((L * BINS,), jnp.float32),
            pltpu.VMEM((BINS,), jnp.float32),
            pltpu.SemaphoreType.DMA,
            pltpu.SemaphoreType.DMA,
            pltpu.SemaphoreType.DMA,
        ],
    )


def _tc_finish_body(parts_ref, w1_ref, r1_ref, b1_ref, w2_ref, rt2_ref,
                    b2_ref, out_ref):
    counts = jnp.sum(parts_ref[...], axis=0)
    g = (lax.broadcasted_iota(jnp.int32, (RS, R), 0) // N
         == lax.broadcasted_iota(jnp.int32, (RS, R), 1)).astype(jnp.float32)
    gt = (lax.broadcasted_iota(jnp.int32, (R, RS), 1) // N
          == lax.broadcasted_iota(jnp.int32, (R, RS), 0)).astype(jnp.float32)
    cnt = jnp.dot(counts, g, preferred_element_type=jnp.float32)
    denom = jnp.maximum(
        jnp.dot(cnt, gt, preferred_element_type=jnp.float32), 1.0)
    q = counts / denom
    agg1 = jnp.dot(q, w1_ref[...], preferred_element_type=jnp.float32)
    h = jnp.maximum(agg1 + r1_ref[...] + b1_ref[...], 0.0)
    w2h = jnp.concatenate(
        [jnp.dot(h, w2_ref[r], preferred_element_type=jnp.float32)
         for r in range(R)], axis=0)
    acc = (jnp.dot(q, w2h, preferred_element_type=jnp.float32)
           + jnp.dot(h, rt2_ref[...], preferred_element_type=jnp.float32)
           + b2_ref[...])
    m = jnp.max(acc, axis=1, keepdims=True)
    e = jnp.exp(acc - m)
    lse = jnp.log(jnp.sum(e, axis=1, keepdims=True))
    out_ref[...] = acc - m - lse


def kernel(x, edge_index, edge_type, weight1, root1, bias1, weight2, root2,
           bias2):
    del x
    parts = _make_sc_hist()(edge_index.reshape(2 * E), edge_type)
    parts = parts.reshape(NW, N, RS)
    return pl.pallas_call(
        _tc_finish_body,
        out_shape=jax.ShapeDtypeStruct((N, OUT), jnp.float32),
    )(parts, weight1.reshape(RS, HID), root1, bias1.reshape(1, HID),
      weight2, root2, bias2.reshape(1, OUT))
